# baseline (speedup 1.0000x reference)
import jax
import jax.numpy as jnp
from jax.experimental import pallas as pl
from jax.experimental.pallas import tpu as pltpu

N_NODES = 12
FANOUT = 4
EPSILON = 0.01
EPS_BN = 1e-5
C = 192
EMB = 64
B = 128
H = 14
W = 14
HW = H * W
BB = 16
NBLK = B // BB
CH = 28
NSTEP = N_NODES - 1


def _mega_body(x_ref, wtT_ref, P_ref, embsT_ref, rm_ref, rv_ref, g_ref,
               b_ref, out_ref, ring, ad_s, cf_s, cn_s, psum, stage, dsem):
    i = pl.program_id(0)
    sub = pl.program_id(1)
    gb = jnp.minimum(sub, NBLK - 1) * BB

    @pl.when((i == 0) & (sub < NBLK))
    def _():
        cp = pltpu.make_async_copy(x_ref.at[:, pl.ds(gb, BB), :], stage, dsem)
        cp.start()
        cp.wait()
        xb = stage[...]
        ring[0, :, pl.ds(gb, BB), :] = xb.astype(jnp.bfloat16)
        psum[pl.ds(gb, BB), :] = jnp.sum(xb, axis=0)

    for istat in (1, 2, 3):
        @pl.when((i == istat) & (sub < NBLK))
        def _(istat=istat):
            agg = jnp.zeros((HW, BB, C), jnp.float32)
            for p in range(istat):
                c = cn_s[pl.ds(p, 1), pl.ds(gb, BB), :]
                t = ring[p % 4, :, pl.ds(gb, BB), :].astype(jnp.float32)
                agg = agg + c * t
            a2 = agg.reshape(HW * BB, C)
            y = jnp.dot(a2, wtT_ref[istat - 1],
                        preferred_element_type=jnp.float32)
            y = jnp.maximum(y, 0.0).reshape(HW, BB, C)
            ring[istat % 4, :, pl.ds(gb, BB), :] = y.astype(jnp.bfloat16)
            psum[pl.ds(gb, BB), :] = jnp.sum(y, axis=0)

    @pl.when((i >= FANOUT) & (i <= NSTEP - 1) & (sub < NBLK))
    def _():
        agg = jnp.zeros((HW, BB, C), jnp.float32)
        for o in range(1, FANOUT + 1):
            pc = jnp.maximum(i - o, 0)
            slot = jax.lax.rem(pc, 4)
            c = cn_s[pl.ds(pc, 1), pl.ds(gb, BB), :]
            t = ring[slot, :, pl.ds(gb, BB), :].astype(jnp.float32)
            agg = agg + c * t
        a2 = agg.reshape(HW * BB, C)
        y = jnp.dot(a2, wtT_ref[i - 1], preferred_element_type=jnp.float32)
        y = jnp.maximum(y, 0.0).reshape(HW, BB, C)
        ring[jax.lax.rem(i, 4), :, pl.ds(gb, BB), :] = y.astype(jnp.bfloat16)
        psum[pl.ds(gb, BB), :] = jnp.sum(y, axis=0)

    @pl.when((i <= NSTEP - 1) & (sub == NBLK))
    def _():
        pooled = psum[...] * (1.0 / HW)
        q = jnp.dot(pooled, P_ref[...], preferred_element_type=jnp.float32)
        sc = jnp.dot(q, embsT_ref[...], preferred_element_type=jnp.float32)
        sc = (sc - rm_ref[...]) / jnp.sqrt(rv_ref[...] + EPS_BN) \
            * g_ref[...] + b_ref[...]
        lane = jax.lax.broadcasted_iota(jnp.int32, (B, N_NODES), 1)
        win = (lane >= i + 1) & (lane <= jnp.minimum(i + FANOUT, N_NODES - 1))
        scm = jnp.where(win, sc, -1e30)
        mx = jnp.max(scm, axis=1, keepdims=True)
        e = jnp.where(win, jnp.exp(scm - mx), 0.0)
        trans = e / jnp.sum(e, axis=1, keepdims=True)
        ad = jnp.where(i == 0, 0.0, ad_s[...])
        a = jnp.where(i == 0, 1.0,
                      jnp.sum(jnp.where(lane == i, ad, 0.0), axis=1,
                              keepdims=True))
        sent = a * trans
        ad1 = jnp.where(lane == i, 0.0, ad) + sent
        mk = (ad1 > EPSILON).astype(jnp.float32)
        s = 1.0 / (jnp.sum(ad1 * mk, axis=1, keepdims=True) + 1e-12)
        mrow = mk * s
        ad_s[...] = ad1 * mrow
        cf = jnp.where(i == 0, 0.0, cf_s[...])
        psub = jax.lax.broadcasted_iota(jnp.int32, (N_NODES, B, N_NODES), 0)
        cf = cf + jnp.where(psub == i, sent[None, :, :], 0.0)
        cf = cf * mrow[None, :, :]
        cf_s[...] = cf
        lane3 = jax.lax.broadcasted_iota(jnp.int32, (N_NODES, B, N_NODES), 2)
        cn_s[...] = jnp.sum(jnp.where(lane3 == i + 1, cf, 0.0), axis=2,
                            keepdims=True)

    @pl.when((i == NSTEP) & (sub < NBLK))
    def _():
        acc = jnp.zeros((HW, BB, C), jnp.float32)
        for p in range(N_NODES - 1 - FANOUT, N_NODES - 1):
            c = cn_s[pl.ds(p, 1), pl.ds(gb, BB), :]
            t = ring[p % 4, :, pl.ds(gb, BB), :].astype(jnp.float32)
            acc = acc + c * t
        stage[...] = acc
        cp = pltpu.make_async_copy(stage, out_ref.at[:, pl.ds(gb, BB), :],
                                   dsem)
        cp.start()
        cp.wait()


def kernel(x, node_embs, node_gamma, node_beta, running_mean, running_var,
           Wt, P):
    f32 = jnp.float32
    xr = jnp.transpose(x, (2, 3, 0, 1)).reshape(HW, B, C)
    WtT = jnp.transpose(Wt[1:N_NODES - 1], (0, 2, 1))
    embsT = node_embs.T
    rm = running_mean.reshape(1, N_NODES)
    rv = running_var.reshape(1, N_NODES)
    g = node_gamma.reshape(1, N_NODES)
    b = node_beta.reshape(1, N_NODES)

    def c0(shape):
        nd = len(shape)
        return pl.BlockSpec(shape, lambda i, s: (0,) * nd)

    out2d = pl.pallas_call(
        _mega_body,
        grid=(NSTEP + 1, NBLK + 1),
        in_specs=[
            pl.BlockSpec(memory_space=pl.ANY),
            c0((N_NODES - 2, C, C)),
            c0((C, EMB)), c0((EMB, N_NODES)),
            c0((1, N_NODES)), c0((1, N_NODES)),
            c0((1, N_NODES)), c0((1, N_NODES)),
        ],
        out_specs=pl.BlockSpec(memory_space=pl.ANY),
        out_shape=jax.ShapeDtypeStruct((HW, B, C), f32),
        scratch_shapes=[
            pltpu.VMEM((4, HW, B, C), jnp.bfloat16),
            pltpu.VMEM((B, N_NODES), f32),
            pltpu.VMEM((N_NODES, B, N_NODES), f32),
            pltpu.VMEM((N_NODES, B, 1), f32),
            pltpu.VMEM((B, C), f32),
            pltpu.VMEM((HW, BB, C), f32),
            pltpu.SemaphoreType.DMA,
        ],
        compiler_params=pltpu.CompilerParams(
            vmem_limit_bytes=64 * 1024 * 1024),
    )(xr, WtT, P, embsT, rm, rv, g, b)

    return out2d.reshape(H, W, B, C).transpose(2, 3, 0, 1)


# speedup vs baseline: 1.0541x; 1.0541x over previous
import jax
import jax.numpy as jnp
from jax.experimental import pallas as pl
from jax.experimental.pallas import tpu as pltpu

N_NODES = 12
FANOUT = 4
EPSILON = 0.01
EPS_BN = 1e-5
C = 192
EMB = 64
B = 128
H = 14
W = 14
HW = H * W
BB = 16
NPAIR = 2
SB = BB * NPAIR
NBLK = B // SB
CH = 28
NSTEP = N_NODES - 1


def _mega_body(x_ref, wtT_ref, P_ref, embsT_ref, rm_ref, rv_ref, g_ref,
               b_ref, out_ref, ring, ad_s, cf_s, cn_s, psum, stage, dsem):
    i = pl.program_id(0)
    sub = pl.program_id(1)
    gb0 = jnp.minimum(sub, NBLK - 1) * SB

    @pl.when((i == 0) & (sub < NBLK))
    def _():
        for k in range(NPAIR):
            gb = gb0 + k * BB
            cp = pltpu.make_async_copy(x_ref.at[:, pl.ds(gb, BB), :], stage,
                                       dsem)
            cp.start()
            cp.wait()
            xb = stage[...]
            ring[0, :, pl.ds(gb, BB), :] = xb.astype(jnp.bfloat16)
            psum[pl.ds(gb, BB), :] = jnp.sum(xb, axis=0)

    for istat in (1, 2, 3):
        @pl.when((i == istat) & (sub < NBLK))
        def _(istat=istat):
            for k in range(NPAIR):
                gb = gb0 + k * BB
                agg = jnp.zeros((HW, BB, C), jnp.float32)
                for p in range(istat):
                    c = cn_s[pl.ds(p, 1), pl.ds(gb, BB), :]
                    t = ring[p % 4, :, pl.ds(gb, BB), :].astype(jnp.float32)
                    agg = agg + c * t
                a2 = agg.reshape(HW * BB, C)
                y = jnp.dot(a2, wtT_ref[istat - 1],
                            preferred_element_type=jnp.float32)
                y = jnp.maximum(y, 0.0).reshape(HW, BB, C)
                ring[istat % 4, :, pl.ds(gb, BB), :] = y.astype(jnp.bfloat16)
                psum[pl.ds(gb, BB), :] = jnp.sum(y, axis=0)

    @pl.when((i >= FANOUT) & (i <= NSTEP - 1) & (sub < NBLK))
    def _():
        for k in range(NPAIR):
            gb = gb0 + k * BB
            agg = jnp.zeros((HW, BB, C), jnp.float32)
            for o in range(1, FANOUT + 1):
                pc = jnp.maximum(i - o, 0)
                slot = jax.lax.rem(pc, 4)
                c = cn_s[pl.ds(pc, 1), pl.ds(gb, BB), :]
                t = ring[slot, :, pl.ds(gb, BB), :].astype(jnp.float32)
                agg = agg + c * t
            a2 = agg.reshape(HW * BB, C)
            y = jnp.dot(a2, wtT_ref[i - 1],
                        preferred_element_type=jnp.float32)
            y = jnp.maximum(y, 0.0).reshape(HW, BB, C)
            ring[jax.lax.rem(i, 4), :, pl.ds(gb, BB), :] = \
                y.astype(jnp.bfloat16)
            psum[pl.ds(gb, BB), :] = jnp.sum(y, axis=0)

    @pl.when((i <= NSTEP - 1) & (sub == NBLK))
    def _():
        pooled = psum[...] * (1.0 / HW)
        q = jnp.dot(pooled, P_ref[...], preferred_element_type=jnp.float32)
        sc = jnp.dot(q, embsT_ref[...], preferred_element_type=jnp.float32)
        sc = (sc - rm_ref[...]) / jnp.sqrt(rv_ref[...] + EPS_BN) \
            * g_ref[...] + b_ref[...]
        lane = jax.lax.broadcasted_iota(jnp.int32, (B, N_NODES), 1)
        win = (lane >= i + 1) & (lane <= jnp.minimum(i + FANOUT, N_NODES - 1))
        scm = jnp.where(win, sc, -1e30)
        mx = jnp.max(scm, axis=1, keepdims=True)
        e = jnp.where(win, jnp.exp(scm - mx), 0.0)
        trans = e / jnp.sum(e, axis=1, keepdims=True)
        ad = jnp.where(i == 0, 0.0, ad_s[...])
        a = jnp.where(i == 0, 1.0,
                      jnp.sum(jnp.where(lane == i, ad, 0.0), axis=1,
                              keepdims=True))
        sent = a * trans
        ad1 = jnp.where(lane == i, 0.0, ad) + sent
        mk = (ad1 > EPSILON).astype(jnp.float32)
        s = 1.0 / (jnp.sum(ad1 * mk, axis=1, keepdims=True) + 1e-12)
        mrow = mk * s
        ad_s[...] = ad1 * mrow
        cf = jnp.where(i == 0, 0.0, cf_s[...])
        psub = jax.lax.broadcasted_iota(jnp.int32, (N_NODES, B, N_NODES), 0)
        cf = cf + jnp.where(psub == i, sent[None, :, :], 0.0)
        cf = cf * mrow[None, :, :]
        cf_s[...] = cf
        lane3 = jax.lax.broadcasted_iota(jnp.int32, (N_NODES, B, N_NODES), 2)
        cn_s[...] = jnp.sum(jnp.where(lane3 == i + 1, cf, 0.0), axis=2,
                            keepdims=True)

    @pl.when((i == NSTEP) & (sub < NBLK))
    def _():
        for k in range(NPAIR):
            gb = gb0 + k * BB
            acc = jnp.zeros((HW, BB, C), jnp.float32)
            for p in range(N_NODES - 1 - FANOUT, N_NODES - 1):
                c = cn_s[pl.ds(p, 1), pl.ds(gb, BB), :]
                t = ring[p % 4, :, pl.ds(gb, BB), :].astype(jnp.float32)
                acc = acc + c * t
            stage[...] = acc
            cp = pltpu.make_async_copy(stage,
                                       out_ref.at[:, pl.ds(gb, BB), :], dsem)
            cp.start()
            cp.wait()


def kernel(x, node_embs, node_gamma, node_beta, running_mean, running_var,
           Wt, P):
    f32 = jnp.float32
    xr = jnp.transpose(x, (2, 3, 0, 1)).reshape(HW, B, C)
    WtT = jnp.transpose(Wt[1:N_NODES - 1], (0, 2, 1))
    embsT = node_embs.T
    rm = running_mean.reshape(1, N_NODES)
    rv = running_var.reshape(1, N_NODES)
    g = node_gamma.reshape(1, N_NODES)
    b = node_beta.reshape(1, N_NODES)

    def c0(shape):
        nd = len(shape)
        return pl.BlockSpec(shape, lambda i, s: (0,) * nd)

    out2d = pl.pallas_call(
        _mega_body,
        grid=(NSTEP + 1, NBLK + 1),
        in_specs=[
            pl.BlockSpec(memory_space=pl.ANY),
            c0((N_NODES - 2, C, C)),
            c0((C, EMB)), c0((EMB, N_NODES)),
            c0((1, N_NODES)), c0((1, N_NODES)),
            c0((1, N_NODES)), c0((1, N_NODES)),
        ],
        out_specs=pl.BlockSpec(memory_space=pl.ANY),
        out_shape=jax.ShapeDtypeStruct((HW, B, C), f32),
        scratch_shapes=[
            pltpu.VMEM((4, HW, B, C), jnp.bfloat16),
            pltpu.VMEM((B, N_NODES), f32),
            pltpu.VMEM((N_NODES, B, N_NODES), f32),
            pltpu.VMEM((N_NODES, B, 1), f32),
            pltpu.VMEM((B, C), f32),
            pltpu.VMEM((HW, BB, C), f32),
            pltpu.SemaphoreType.DMA,
        ],
        compiler_params=pltpu.CompilerParams(
            vmem_limit_bytes=64 * 1024 * 1024),
    )(xr, WtT, P, embsT, rm, rv, g, b)

    return out2d.reshape(H, W, B, C).transpose(2, 3, 0, 1)
